# baseline (speedup 1.0000x reference)
import jax
import jax.numpy as jnp
from jax import lax
from jax.experimental import pallas as pl
from jax.experimental.pallas import tpu as pltpu
from jax.experimental.pallas import tpu_sc as plsc

E = 320000
V = 10000
D = 128
NC = 2
NS = 16
L = 16
NW = NC * NS
EW = E // NW
CHB = 80
NCHUNK = EW // CHB
NBUF = 4
NGB = CHB // L


def _body(xu, xr, iu, ir, out, idxu_v, idxr_v, urows, rrows, pbuf,
          out_all, sem):
    c = lax.axis_index("c")
    s = lax.axis_index("s")
    wid = s * NC + c
    base_w = wid * EW

    pltpu.sync_copy(iu.at[pl.ds(base_w, EW)], idxu_v)
    pltpu.sync_copy(ir.at[pl.ds(base_w, EW)], idxr_v)

    def issue(ci, b):
        pltpu.async_copy(
            xu.at[idxu_v.at[pl.ds(ci * CHB, CHB)]], urows.at[b], sem.at[b])
        pltpu.async_copy(
            xr.at[idxr_v.at[pl.ds(ci * CHB, CHB)]], rrows.at[b], sem.at[b])

    for b in range(NBUF):
        issue(b, b)

    def chunk_body(ci, carry):
        b = lax.rem(ci, NBUF)
        pltpu.make_async_copy(xu.at[pl.ds(0, CHB)], urows.at[b],
                              sem.at[b]).wait()
        pltpu.make_async_copy(xu.at[pl.ds(0, CHB)], rrows.at[b],
                              sem.at[b]).wait()

        def group_body(g, gcarry):
            for i in range(L):
                e = g * L + i
                p = None
                for k in range(D // (2 * L)):
                    uv = plsc.bitcast(urows[b, e, pl.ds(k * L, L)], jnp.bfloat16)
                    rv = plsc.bitcast(rrows[b, e, pl.ds(k * L, L)], jnp.bfloat16)
                    t = uv * rv
                    p = t if p is None else p + t
                pa, pb = plsc.unpack(p, format=plsc.PackFormat.INTERLEAVED)
                pbuf[pl.ds(i * L, L)] = pa + pb
            ebase = lax.iota(jnp.int32, L) * L
            acc = plsc.load_gather(pbuf, [ebase])
            for j in range(1, L):
                acc = acc + plsc.load_gather(pbuf, [ebase + j])
            out_all[pl.ds(ci * CHB + g * L, L)] = acc
            return gcarry

        lax.fori_loop(0, NGB, group_body, 0, unroll=False)

        @pl.when(ci < NCHUNK - NBUF)
        def _():
            issue(ci + NBUF, b)

        return carry

    lax.fori_loop(0, NCHUNK, chunk_body, 0, unroll=False)
    pltpu.sync_copy(out_all, out.at[pl.ds(base_w, EW)])


@jax.jit
def _run(xu, xr, iu, ir):
    mesh = plsc.VectorSubcoreMesh(
        core_axis_name="c", subcore_axis_name="s", num_cores=NC,
        num_subcores=NS)
    return pl.kernel(
        _body,
        out_type=jax.ShapeDtypeStruct((E,), jnp.float32),
        mesh=mesh,
        compiler_params=pltpu.CompilerParams(needs_layout_passes=False, use_tc_tiling_on_sc=False),
        scratch_types=[
            pltpu.VMEM((EW,), jnp.int32),
            pltpu.VMEM((EW,), jnp.int32),
            pltpu.VMEM((NBUF, CHB, D // 2), jnp.int32),
            pltpu.VMEM((NBUF, CHB, D // 2), jnp.int32),
            pltpu.VMEM((L * L,), jnp.float32),
            pltpu.VMEM((EW,), jnp.float32),
            pltpu.SemaphoreType.DMA((NBUF,)),
        ],
    )(xu, xr, iu, ir)


def _pack_bf16(x):
    xb = x.astype(jnp.bfloat16).reshape(x.shape[0], x.shape[1] // 2, 2)
    return jax.lax.bitcast_convert_type(xb, jnp.int32)


def kernel(x_user, x_restaurant, edge_label_index):
    eli = edge_label_index.astype(jnp.int32)
    return _run(_pack_bf16(x_user), _pack_bf16(x_restaurant), eli[0], eli[1])


# speedup vs baseline: 1.1701x; 1.1701x over previous
import jax
import jax.numpy as jnp
from jax import lax
from jax.experimental import pallas as pl
from jax.experimental.pallas import tpu as pltpu
from jax.experimental.pallas import tpu_sc as plsc

E = 320000
V = 10000
D = 128
NC = 2
NS = 16
L = 16
NW = NC * NS
EW = E // NW
CHB = 80
NCHUNK = EW // CHB
NBUF = 4
NGB = CHB // L


def _body(xu, xr, iu, ir, out, idxu_v, idxr_v, urows, rrows, pbuf,
          out_all, sem):
    c = lax.axis_index("c")
    s = lax.axis_index("s")
    wid = s * NC + c
    base_w = wid * EW

    pltpu.sync_copy(iu.at[pl.ds(base_w, EW)], idxu_v)
    pltpu.sync_copy(ir.at[pl.ds(base_w, EW)], idxr_v)

    def issue(ci, b):
        pltpu.async_copy(
            xu.at[idxu_v.at[pl.ds(ci * CHB, CHB)]], urows.at[b], sem.at[b])
        pltpu.async_copy(
            xr.at[idxr_v.at[pl.ds(ci * CHB, CHB)]], rrows.at[b], sem.at[b])

    for b in range(NBUF):
        issue(b, b)

    def chunk_body(ci, carry):
        b = lax.rem(ci, NBUF)
        pltpu.make_async_copy(xu.at[pl.ds(0, CHB)], urows.at[b],
                              sem.at[b]).wait()
        pltpu.make_async_copy(xu.at[pl.ds(0, CHB)], rrows.at[b],
                              sem.at[b]).wait()

        def group_body(g, gcarry):
            for i in range(L):
                e = g * L + i
                p = None
                for k in range(D // L):
                    t = urows[b, e, pl.ds(k * L, L)] * rrows[b, e, pl.ds(k * L, L)]
                    p = t if p is None else p + t
                pbuf[pl.ds(i * L, L)] = p
            ebase = lax.iota(jnp.int32, L) * L
            acc = plsc.load_gather(pbuf, [ebase])
            for j in range(1, L):
                acc = acc + plsc.load_gather(pbuf, [ebase + j])
            out_all[pl.ds(ci * CHB + g * L, L)] = acc
            return gcarry

        lax.fori_loop(0, NGB, group_body, 0, unroll=False)

        @pl.when(ci < NCHUNK - NBUF)
        def _():
            issue(ci + NBUF, b)

        return carry

    lax.fori_loop(0, NCHUNK, chunk_body, 0, unroll=False)
    pltpu.sync_copy(out_all, out.at[pl.ds(base_w, EW)])


@jax.jit
def _run(xu, xr, iu, ir):
    mesh = plsc.VectorSubcoreMesh(
        core_axis_name="c", subcore_axis_name="s", num_cores=NC,
        num_subcores=NS)
    return pl.kernel(
        _body,
        out_type=jax.ShapeDtypeStruct((E,), jnp.float32),
        mesh=mesh,
        compiler_params=pltpu.CompilerParams(needs_layout_passes=False),
        scratch_types=[
            pltpu.VMEM((EW,), jnp.int32),
            pltpu.VMEM((EW,), jnp.int32),
            pltpu.VMEM((NBUF, CHB, D), jnp.float32),
            pltpu.VMEM((NBUF, CHB, D), jnp.float32),
            pltpu.VMEM((L * L,), jnp.float32),
            pltpu.VMEM((EW,), jnp.float32),
            pltpu.SemaphoreType.DMA((NBUF,)),
        ],
    )(xu, xr, iu, ir)


def kernel(x_user, x_restaurant, edge_label_index):
    eli = edge_label_index.astype(jnp.int32)
    return _run(x_user, x_restaurant, eli[0], eli[1])
